# baseline (speedup 1.0000x reference)
import functools

import jax
import jax.numpy as jnp
from jax import lax
from jax.experimental import pallas as pl
from jax.experimental.pallas import tpu as pltpu
from jax.experimental.pallas import tpu_sc as plsc

_NC = 2
_NS = 16
_NW = _NC * _NS
_LANES = 16
_CHUNK = 32


def _tc_add_kernel(emb_ref, m0_ref, m1_ref, o0_ref, o1_ref):
    o0_ref[...] = m0_ref[...] + emb_ref[0:1, :]
    o1_ref[...] = m1_ref[...] + emb_ref[1:2, :]


def _sc_add_body(x_hbm, emb_hbm, out_hbm, buf, emb_v, sem):
    D = emb_hbm.shape[0]
    n_rows = x_hbm.shape[0]
    rows_per_w = n_rows // _NW
    wid = lax.axis_index("s") * _NC + lax.axis_index("c")
    base = wid * rows_per_w
    pltpu.sync_copy(emb_hbm, emb_v)

    n_chunks = rows_per_w // _CHUNK

    def chunk_body(ci, _):
        r0 = base + ci * _CHUNK
        pltpu.sync_copy(x_hbm.at[pl.ds(r0, _CHUNK)], buf)

        def row_body(r, _):
            for j in range(D // _LANES):
                s = pl.ds(j * _LANES, _LANES)
                buf[r, s] = buf[r, s] + emb_v[s]
            return 0

        lax.fori_loop(0, _CHUNK, row_body, 0)
        pltpu.sync_copy(buf, out_hbm.at[pl.ds(r0, _CHUNK)])
        return 0

    lax.fori_loop(0, n_chunks, chunk_body, 0)


def _sc_add(x, emb_row):
    n_rows, D = x.shape
    mesh = plsc.VectorSubcoreMesh(core_axis_name="c", subcore_axis_name="s")
    f = functools.partial(
        pl.kernel,
        mesh=mesh,
        out_type=jax.ShapeDtypeStruct((n_rows, D), jnp.float32),
        scratch_types=[
            pltpu.VMEM((_CHUNK, D), jnp.float32),
            pltpu.VMEM((D,), jnp.float32),
            pltpu.SemaphoreType.DMA,
        ],
    )(_sc_add_body)
    return f(x, emb_row)


def kernel(mod0, mod1, mod2, emb):
    B, L, D = mod0.shape
    N = B * L
    R = 1024
    x0 = mod0.reshape(N, D)
    x1 = mod1.reshape(N, D)
    x2 = mod2.reshape(N, D)

    out2 = _sc_add(x2, emb[2])

    row_spec = pl.BlockSpec((R, D), lambda i: (i, 0))
    out0, out1 = pl.pallas_call(
        _tc_add_kernel,
        grid=(N // R,),
        in_specs=[
            pl.BlockSpec((emb.shape[0], D), lambda i: (0, 0)),
            row_spec, row_spec,
        ],
        out_specs=[row_spec, row_spec],
        out_shape=[jax.ShapeDtypeStruct((N, D), jnp.float32)] * 2,
    )(emb, x0, x1)

    return (out0.reshape(B, L, D), out1.reshape(B, L, D),
            out2.reshape(B, L, D))


# speedup vs baseline: 1.1409x; 1.1409x over previous
import functools

import jax
import jax.numpy as jnp
from jax import lax
from jax.experimental import pallas as pl
from jax.experimental.pallas import tpu as pltpu
from jax.experimental.pallas import tpu_sc as plsc

_NC = 2
_NS = 16
_NW = _NC * _NS
_LANES = 16
_CHUNK = 32
_NBUF = 3


def _tc_add_kernel(emb_ref, m0_ref, m1_ref, o0_ref, o1_ref):
    o0_ref[...] = m0_ref[...] + emb_ref[0:1, :]
    o1_ref[...] = m1_ref[...] + emb_ref[1:2, :]


def _sc_add_body(x_hbm, emb_hbm, out_hbm,
                 b0, b1, b2, emb_v,
                 l0, l1, l2, s0, s1, s2):
    D = emb_hbm.shape[0]
    n_rows = x_hbm.shape[0]
    rows_per_w = n_rows // _NW
    wid = lax.axis_index("s") * _NC + lax.axis_index("c")
    base = wid * rows_per_w
    pltpu.sync_copy(emb_hbm, emb_v)

    bufs = (b0, b1, b2)
    lsems = (l0, l1, l2)
    ssems = (s0, s1, s2)
    n_chunks = rows_per_w // _CHUNK
    loads = [None] * _NBUF
    stores = [None] * _NBUF

    def _load(ci):
        q = ci % _NBUF
        loads[q] = pltpu.async_copy(
            x_hbm.at[pl.ds(base + ci * _CHUNK, _CHUNK)], bufs[q], lsems[q])

    for ci in range(min(_NBUF - 1, n_chunks)):
        _load(ci)

    for ci in range(n_chunks):
        q = ci % _NBUF
        buf = bufs[q]
        loads[q].wait()

        def row_body(r, _):
            for j in range(D // _LANES):
                s = pl.ds(j * _LANES, _LANES)
                buf[r, s] = buf[r, s] + emb_v[s]
            return 0

        lax.fori_loop(0, _CHUNK, row_body, 0)
        stores[q] = pltpu.async_copy(
            buf, out_hbm.at[pl.ds(base + ci * _CHUNK, _CHUNK)], ssems[q])

        nxt = ci + _NBUF - 1
        if nxt < n_chunks:
            qn = nxt % _NBUF
            if stores[qn] is not None:
                stores[qn].wait()
            _load(nxt)

    for q in range(_NBUF):
        if stores[q] is not None:
            stores[q].wait()


def _sc_add(x, emb_row):
    n_rows, D = x.shape
    mesh = plsc.VectorSubcoreMesh(core_axis_name="c", subcore_axis_name="s")
    f = functools.partial(
        pl.kernel,
        mesh=mesh,
        out_type=jax.ShapeDtypeStruct((n_rows, D), jnp.float32),
        scratch_types=[
            pltpu.VMEM((_CHUNK, D), jnp.float32),
            pltpu.VMEM((_CHUNK, D), jnp.float32),
            pltpu.VMEM((_CHUNK, D), jnp.float32),
            pltpu.VMEM((D,), jnp.float32),
            pltpu.SemaphoreType.DMA,
            pltpu.SemaphoreType.DMA,
            pltpu.SemaphoreType.DMA,
            pltpu.SemaphoreType.DMA,
            pltpu.SemaphoreType.DMA,
            pltpu.SemaphoreType.DMA,
        ],
    )(_sc_add_body)
    return f(x, emb_row)


def kernel(mod0, mod1, mod2, emb):
    B, L, D = mod0.shape
    N = B * L
    R = 1024
    x0 = mod0.reshape(N, D)
    x1 = mod1.reshape(N, D)
    x2 = mod2.reshape(N, D)

    out2 = _sc_add(x2, emb[2])

    row_spec = pl.BlockSpec((R, D), lambda i: (i, 0))
    out0, out1 = pl.pallas_call(
        _tc_add_kernel,
        grid=(N // R,),
        in_specs=[
            pl.BlockSpec((emb.shape[0], D), lambda i: (0, 0)),
            row_spec, row_spec,
        ],
        out_specs=[row_spec, row_spec],
        out_shape=[jax.ShapeDtypeStruct((N, D), jnp.float32)] * 2,
    )(emb, x0, x1)

    return (out0.reshape(B, L, D), out1.reshape(B, L, D),
            out2.reshape(B, L, D))
